# Initial kernel scaffold; baseline (speedup 1.0000x reference)
import jax
import jax.numpy as jnp
from jax.experimental import pallas as pl


def kernel(edge_index, user_emb, item_emb, W1, b1, W2, b2):
    raise NotImplementedError("write your pallas kernel here")


# speedup vs baseline: 2.6262x; 2.6262x over previous
import functools

import jax
import jax.numpy as jnp
from jax import lax
from jax.experimental import pallas as pl
from jax.experimental.pallas import tpu as pltpu
from jax.experimental.pallas import tpu_sc as plsc

N = 100000
NP = 100096
D = 32
E = 1600000
G = 12544
EP = G * 128
GP32 = G // 32
KH = 8
ITERS = GP32 // KH

F32 = jnp.float32


def _hist_body(e0, e1, out, cnt, idxb, sem):
    c = lax.axis_index("c")
    s = lax.axis_index("s")
    w = s * 2 + c
    ones16 = jnp.full((16,), 1.0, F32)

    def it_body(it, carry):
        @pl.when(it == 0)
        def _():
            def zb(i, c2):
                cnt[pl.ds(i * 16, 16)] = jnp.zeros((16,), F32)
                return c2

            lax.fori_loop(0, NP // 16, zb, 0)

        for e_ref in (e0, e1):
            pltpu.sync_copy(e_ref.at[pl.ds(w * GP32 + it * KH, KH)], idxb)
            for j in range(KH):
                for k in range(8):
                    v = idxb[j, pl.ds(k * 16, 16)]
                    plsc.addupdate_scatter(cnt, [v], ones16)
        return carry

    lax.fori_loop(0, ITERS, it_body, 0)
    pltpu.sync_copy(cnt, out.at[w])


@functools.lru_cache(maxsize=1)
def _sc_kernels():
    mesh = plsc.VectorSubcoreMesh(core_axis_name="c", subcore_axis_name="s")
    hist = pl.kernel(
        _hist_body,
        out_type=jax.ShapeDtypeStruct((32, NP), F32),
        mesh=mesh,
        compiler_params=pltpu.CompilerParams(needs_layout_passes=False),
        scratch_types=[
            pltpu.VMEM((NP,), F32),
            pltpu.VMEM((KH, 128), jnp.int32),
            pltpu.SemaphoreType.DMA,
        ],
    )
    return hist


R = 2000
_GRID = N // R
BC = 4352


def _tc_deg_body(p_r, deg_r):
    red = jnp.sum(p_r[...], axis=0) + 1.0
    deg_r[...] = jnp.broadcast_to(red[:, None], (BC, 16))


_tc_deg = pl.pallas_call(
    _tc_deg_body,
    grid=(NP // BC,),
    in_specs=[pl.BlockSpec((32, BC), lambda i: (0, i))],
    out_specs=pl.BlockSpec((BC, 16), lambda i: (i, 0)),
    out_shape=jax.ShapeDtypeStruct((NP, 16), F32),
)


def _tc_pre_body(deg_r, x_r, w1_r, ya_r, yb_r, dv_r):
    dinv = lax.rsqrt(deg_r[...][:, :1])
    h = jnp.dot(x_r[...], w1_r[...], preferred_element_type=F32)
    y = h * dinv
    ya_r[...] = y[:, :16]
    yb_r[...] = y[:, 16:]
    dv_r[...] = jnp.broadcast_to(dinv, (R, 16))


_tc_pre = pl.pallas_call(
    _tc_pre_body,
    grid=(_GRID,),
    in_specs=[
        pl.BlockSpec((R, 16), lambda i: (i, 0)),
        pl.BlockSpec((R, 32), lambda i: (i, 0)),
        pl.BlockSpec((32, 32), lambda i: (0, 0)),
    ],
    out_specs=[
        pl.BlockSpec((R, 16), lambda i: (i, 0)),
        pl.BlockSpec((R, 16), lambda i: (i, 0)),
        pl.BlockSpec((R, 16), lambda i: (i, 0)),
    ],
    out_shape=[
        jax.ShapeDtypeStruct((NP, 16), F32),
        jax.ShapeDtypeStruct((NP, 16), F32),
        jax.ShapeDtypeStruct((NP, 16), F32),
    ],
)


def _tc_mid_body(za_r, zb_r, ya_r, yb_r, dv_r, w2_r, b1_r, oa_r, ob_r):
    dinv = dv_r[...][:, :1]
    z = jnp.concatenate([za_r[...], zb_r[...]], axis=1)
    y = jnp.concatenate([ya_r[...], yb_r[...]], axis=1)
    x1 = dinv * (z + y) + b1_r[...][0:1, :]
    h2 = jnp.dot(x1, w2_r[...], preferred_element_type=F32)
    y2 = h2 * dinv
    oa_r[...] = y2[:, :16]
    ob_r[...] = y2[:, 16:]


_tc_mid = pl.pallas_call(
    _tc_mid_body,
    grid=(_GRID,),
    in_specs=[
        pl.BlockSpec((R, 16), lambda i: (i, 0)),
        pl.BlockSpec((R, 16), lambda i: (i, 0)),
        pl.BlockSpec((R, 16), lambda i: (i, 0)),
        pl.BlockSpec((R, 16), lambda i: (i, 0)),
        pl.BlockSpec((R, 16), lambda i: (i, 0)),
        pl.BlockSpec((32, 32), lambda i: (0, 0)),
        pl.BlockSpec((8, 32), lambda i: (0, 0)),
    ],
    out_specs=[
        pl.BlockSpec((R, 16), lambda i: (i, 0)),
        pl.BlockSpec((R, 16), lambda i: (i, 0)),
    ],
    out_shape=[
        jax.ShapeDtypeStruct((NP, 16), F32),
        jax.ShapeDtypeStruct((NP, 16), F32),
    ],
)


def _tc_fin_body(za_r, zb_r, ya_r, yb_r, dv_r, b2_r, o_r):
    dinv = dv_r[...][:, :1]
    z = jnp.concatenate([za_r[...], zb_r[...]], axis=1)
    y = jnp.concatenate([ya_r[...], yb_r[...]], axis=1)
    o_r[...] = dinv * (z + y) + b2_r[...][0:1, :]


_tc_fin = pl.pallas_call(
    _tc_fin_body,
    grid=(_GRID,),
    in_specs=[
        pl.BlockSpec((R, 16), lambda i: (i, 0)),
        pl.BlockSpec((R, 16), lambda i: (i, 0)),
        pl.BlockSpec((R, 16), lambda i: (i, 0)),
        pl.BlockSpec((R, 16), lambda i: (i, 0)),
        pl.BlockSpec((R, 16), lambda i: (i, 0)),
        pl.BlockSpec((8, 32), lambda i: (0, 0)),
    ],
    out_specs=pl.BlockSpec((R, 32), lambda i: (i, 0)),
    out_shape=jax.ShapeDtypeStruct((N, 32), F32),
)


def kernel(edge_index, user_emb, item_emb, W1, b1, W2, b2):
    pad = jnp.full((EP - E,), N, dtype=jnp.int32)
    e0 = jnp.concatenate([edge_index[0], pad]).reshape(G, 128)
    e1 = jnp.concatenate([edge_index[1], pad]).reshape(G, 128)
    x = jnp.concatenate([user_emb, item_emb], axis=0)
    b1p = jnp.broadcast_to(b1.reshape(1, 32), (8, 32))
    b2p = jnp.broadcast_to(b2.reshape(1, 32), (8, 32))

    hist = _sc_kernels()
    partials = hist(e0, e1)
    deg16 = _tc_deg(partials)
    ya, yb, dv = _tc_pre(deg16, x, W1)

    src = e0.reshape(-1)
    dst = e1.reshape(-1)

    def spmm(pa, pb):
        y = jnp.concatenate([pa, pb], axis=1)
        z = jnp.zeros((NP, 32), F32)
        z = z.at[dst].add(y[src])
        z = z.at[src].add(y[dst])
        return z[:, :16], z[:, 16:]

    za, zb = spmm(ya, yb)
    y2a, y2b = _tc_mid(za, zb, ya, yb, dv, W2, b1p)
    z2a, z2b = spmm(y2a, y2b)
    return _tc_fin(z2a, z2b, y2a, y2b, dv, b2p)
